# baseline (speedup 1.0000x reference)
import jax
import jax.numpy as jnp
from jax.experimental import pallas as pl
from jax.experimental.pallas import tpu as pltpu
from jax.experimental.pallas import tpu_sc as plsc


_LANES = 16
_BR = 16


def _sc_pos_add(x2d, pos_emb, b, t):
    n, c = x2d.shape
    nblk_t = t // _BR
    spb = t // _BR
    mesh = plsc.VectorSubcoreMesh(core_axis_name="c", subcore_axis_name="s")

    @pl.kernel(out_type=jax.ShapeDtypeStruct((n, c), x2d.dtype), mesh=mesh)
    def run(x_hbm, pe_hbm, o_hbm):
        def body(x_vmem, pe_vmem, o_vmem):
            @pl.loop(0, _BR)
            def _(r):
                @pl.loop(0, c, step=_LANES)
                def _(cc):
                    slc = (pl.ds(r, 1), pl.ds(cc, _LANES))
                    o_vmem.at[*slc][...] = (
                        x_vmem.at[*slc][...] + pe_vmem.at[*slc][...]
                    )

        pltpu.emit_pipeline(
            body,
            grid=(nblk_t, b),
            in_specs=[
                pl.BlockSpec((_BR, c), lambda i, j: (j * spb + i, 0)),
                pl.BlockSpec((_BR, c), lambda i, j: (i, 0)),
            ],
            out_specs=[pl.BlockSpec((_BR, c), lambda i, j: (j * spb + i, 0))],
            core_axis_name=("c", "s"),
            dimension_semantics=(pltpu.PARALLEL, pltpu.PARALLEL),
        )(x_hbm, pe_hbm, o_hbm)

    return run(x2d, pos_emb)


def kernel(x, pos_emb):
    b, t, c = x.shape
    out2d = _sc_pos_add(x.reshape(b * t, c), pos_emb, b, t)
    return out2d.reshape(b, t, c)


# speedup vs baseline: 1.0150x; 1.0150x over previous
import jax
import jax.numpy as jnp
from jax.experimental import pallas as pl
from jax.experimental.pallas import tpu as pltpu
from jax.experimental.pallas import tpu_sc as plsc


_LANES = 16
_BR = 4


def _sc_pos_add(x, pos_emb):
    b, t, c = x.shape
    mesh = plsc.VectorSubcoreMesh(core_axis_name="c", subcore_axis_name="s")

    @pl.kernel(out_type=jax.ShapeDtypeStruct((b, t, c), x.dtype), mesh=mesh)
    def run(x_hbm, pe_hbm, o_hbm):
        def body(x_vmem, pe_vmem, o_vmem):
            @pl.loop(0, b)
            def _(bb):
                @pl.loop(0, _BR)
                def _(r):
                    @pl.loop(0, c, step=_LANES)
                    def _(cc):
                        xs = (pl.ds(bb, 1), pl.ds(r, 1), pl.ds(cc, _LANES))
                        ps = (pl.ds(r, 1), pl.ds(cc, _LANES))
                        o_vmem.at[*xs][...] = (
                            x_vmem.at[*xs][...]
                            + pe_vmem.at[*ps][...].reshape(1, 1, _LANES)
                        )

        pltpu.emit_pipeline(
            body,
            grid=(t // _BR,),
            in_specs=[
                pl.BlockSpec((b, _BR, c), lambda i: (0, i, 0)),
                pl.BlockSpec((_BR, c), lambda i: (i, 0)),
            ],
            out_specs=[pl.BlockSpec((b, _BR, c), lambda i: (0, i, 0))],
            core_axis_name=("c", "s"),
            dimension_semantics=(pltpu.PARALLEL,),
        )(x_hbm, pe_hbm, o_hbm)

    return run(x, pos_emb)


def kernel(x, pos_emb):
    return _sc_pos_add(x, pos_emb)


# speedup vs baseline: 4.5513x; 4.4842x over previous
import jax
import jax.numpy as jnp
from jax.experimental import pallas as pl


def _add_block(x_ref, pe_ref, o_ref):
    o_ref[...] = x_ref[...] + pe_ref[...]


def kernel(x, pos_emb):
    b, t, c = x.shape
    bt = 2048
    grid = (t // bt, b)
    return pl.pallas_call(
        _add_block,
        grid=grid,
        in_specs=[
            pl.BlockSpec((1, bt, c), lambda i, j: (j, i, 0)),
            pl.BlockSpec((bt, c), lambda i, j: (i, 0)),
        ],
        out_specs=pl.BlockSpec((1, bt, c), lambda i, j: (j, i, 0)),
        out_shape=jax.ShapeDtypeStruct((b, t, c), x.dtype),
    )(x, pos_emb)
